# Initial kernel scaffold; baseline (speedup 1.0000x reference)
import jax
import jax.numpy as jnp
from jax.experimental import pallas as pl


def kernel(token_ids, embedding_table):
    raise NotImplementedError("write your pallas kernel here")


# speedup vs baseline: 3.5835x; 3.5835x over previous
import functools

import jax
import jax.numpy as jnp
from jax import lax
from jax.experimental import pallas as pl
from jax.experimental.pallas import tpu as pltpu
from jax.experimental.pallas import tpu_sc as plsc

N_TOKENS = 1000
D = 64
B_ROWS = 4096
B_COLS = 200
B = B_ROWS * B_COLS

NC = 2
NS = 16
NW = NC * NS
BPW = B // NW
CH = 128
NCHUNK = BPW // CH

R = 8
G = 4

_mesh = plsc.VectorSubcoreMesh(core_axis_name="c", subcore_axis_name="s")


@functools.partial(
    pl.kernel,
    mesh=_mesh,
    compiler_params=pltpu.CompilerParams(use_tc_tiling_on_sc=False),
    out_type=jax.ShapeDtypeStruct((NW, NCHUNK, CH, D), jnp.float32),
    scratch_types=[
        pltpu.VMEM((NCHUNK, CH), jnp.int32),
    ]
    + [pltpu.VMEM((CH, D), jnp.float32) for _ in range(R)]
    + [pltpu.SemaphoreType.DMA for _ in range(2 * R)],
)
def _emb_lookup(table_hbm, ids_hbm, out_hbm, idx_v, *rest):
    rows = rest[:R]
    gsem = rest[R:2 * R]
    ssem = rest[2 * R:]

    wid = lax.axis_index("s") * NC + lax.axis_index("c")

    pltpu.sync_copy(ids_hbm.at[wid], idx_v)

    for b in range(G):
        pltpu.async_copy(table_hbm.at[idx_v.at[b]], rows[b], gsem[b])

    def step(j2, _):
        for b in range(R):
            j = j2 * R + b
            pltpu.make_async_copy(table_hbm.at[pl.ds(0, CH)], rows[b], gsem[b]).wait()
            pltpu.async_copy(rows[b], out_hbm.at[wid, j], ssem[b])
            f = j + G
            bf = (b + G) % R

            @pl.when(f < NCHUNK)
            def _():
                @pl.when(j >= R - G)
                def _():
                    pltpu.make_async_copy(
                        rows[bf], out_hbm.at[wid, 0], ssem[bf]).wait()

                pltpu.async_copy(table_hbm.at[idx_v.at[f]], rows[bf], gsem[bf])

        return 0

    lax.fori_loop(0, NCHUNK // R, step, 0)

    for b in range(R):
        pltpu.make_async_copy(rows[b], out_hbm.at[wid, 0], ssem[b]).wait()


def kernel(token_ids, embedding_table):
    ids = token_ids.astype(jnp.int32).reshape(NW, NCHUNK, CH)
    out = _emb_lookup(embedding_table, ids)
    return out.reshape(B_ROWS, B_COLS, D)
